# baseline (speedup 1.0000x reference)
import jax
import jax.numpy as jnp
from jax.experimental import pallas as pl
from jax.experimental.pallas import tpu as pltpu

_BM = 256
_D = 4


def _plane_copy(adj_hbm, abuf, sem, blk, plane):
    slot = blk % _D
    return pltpu.make_async_copy(
        adj_hbm.at[plane, pl.ds(blk * _BM, _BM), :],
        abuf.at[slot, plane],
        sem.at[slot, plane],
    )


def _start_block(adj_hbm, abuf, sem, blk):
    _plane_copy(adj_hbm, abuf, sem, blk, 0).start()
    _plane_copy(adj_hbm, abuf, sem, blk, 1).start()


def _wait_block(adj_hbm, abuf, sem, blk):
    _plane_copy(adj_hbm, abuf, sem, blk, 0).wait()
    _plane_copy(adj_hbm, abuf, sem, blk, 1).wait()


def _manual_kernel(x_ref, w_ref, b_ref, adj_hbm, out_ref, h_ref, abuf, sem):
    i = pl.program_id(0)
    nb = pl.num_programs(0)

    @pl.when(i == 0)
    def _():
        for blk in range(_D - 1):
            _start_block(adj_hbm, abuf, sem, blk)
        din = w_ref.shape[1]
        x = x_ref[...]
        h_ref[0] = jnp.dot(x[:, :din], w_ref[0], preferred_element_type=jnp.float32) + b_ref[0]
        h_ref[1] = jnp.dot(x[:, din:], w_ref[1], preferred_element_type=jnp.float32) + b_ref[1]

    @pl.when(i + _D - 1 < nb)
    def _():
        _start_block(adj_hbm, abuf, sem, i + _D - 1)

    _wait_block(adj_hbm, abuf, sem, i)
    a = abuf[i % _D]
    out_ref[...] = (
        jnp.dot(a[0], h_ref[0], preferred_element_type=jnp.float32)
        + jnp.dot(a[1], h_ref[1], preferred_element_type=jnp.float32)
    )


def kernel(x, adjs, W1, b1, W2, b2):
    n = adjs.shape[1]
    dout = W1.shape[1]
    w = jnp.stack([W1, W2])
    b = jnp.stack([b1, b2]).reshape(2, 1, dout)

    out = pl.pallas_call(
        _manual_kernel,
        grid=(n // _BM,),
        in_specs=[
            pl.BlockSpec((n, x.shape[1]), lambda i: (0, 0)),
            pl.BlockSpec((2, W1.shape[0], dout), lambda i: (0, 0, 0)),
            pl.BlockSpec((2, 1, dout), lambda i: (0, 0, 0)),
            pl.BlockSpec(memory_space=pl.ANY),
        ],
        out_specs=pl.BlockSpec((_BM, dout), lambda i: (i, 0)),
        out_shape=jax.ShapeDtypeStruct((n, dout), jnp.float32),
        scratch_shapes=[
            pltpu.VMEM((2, n, dout), jnp.float32),
            pltpu.VMEM((_D, 2, _BM, n), jnp.float32),
            pltpu.SemaphoreType.DMA((_D, 2)),
        ],
        compiler_params=pltpu.CompilerParams(dimension_semantics=("arbitrary",)),
    )(x, w, b, adjs)
    return out


# speedup vs baseline: 1.0539x; 1.0539x over previous
import jax
import jax.numpy as jnp
from jax.experimental import pallas as pl
from jax.experimental.pallas import tpu as pltpu

_BM = 256


def _agg_kernel(x_ref, w_ref, b_ref, adj_ref, out_ref):
    din = w_ref.shape[1]
    a0 = adj_ref[0]
    a1 = adj_ref[1]
    t0 = jnp.dot(a0, x_ref[:, :din], preferred_element_type=jnp.float32)
    t1 = jnp.dot(a1, x_ref[:, din:], preferred_element_type=jnp.float32)
    r0 = jnp.sum(a0, axis=1, keepdims=True)
    r1 = jnp.sum(a1, axis=1, keepdims=True)
    out_ref[...] = (
        jnp.dot(t0, w_ref[0], preferred_element_type=jnp.float32)
        + jnp.dot(t1, w_ref[1], preferred_element_type=jnp.float32)
        + r0 * b_ref[0]
        + r1 * b_ref[1]
    )


def kernel(x, adjs, W1, b1, W2, b2):
    n = adjs.shape[1]
    dout = W1.shape[1]
    w = jnp.stack([W1, W2])
    b = jnp.stack([b1, b2]).reshape(2, 1, dout)

    out = pl.pallas_call(
        _agg_kernel,
        grid=(n // _BM,),
        in_specs=[
            pl.BlockSpec((n, x.shape[1]), lambda i: (0, 0)),
            pl.BlockSpec((2, W1.shape[0], dout), lambda i: (0, 0, 0)),
            pl.BlockSpec((2, 1, dout), lambda i: (0, 0, 0)),
            pl.BlockSpec((2, _BM, n), lambda i: (0, i, 0)),
        ],
        out_specs=pl.BlockSpec((_BM, dout), lambda i: (i, 0)),
        out_shape=jax.ShapeDtypeStruct((n, dout), jnp.float32),
        compiler_params=pltpu.CompilerParams(dimension_semantics=("parallel",)),
    )(x, w, b, adjs)
    return out
